# baseline (speedup 1.0000x reference)
import functools

import jax
import jax.numpy as jnp
from jax import lax
from jax.experimental import pallas as pl
from jax.experimental.pallas import tpu as pltpu
from jax.experimental.pallas import tpu_sc as plsc

N_NODES = 10000
N_EDGES = 320000
IN_DIM = 128
HIDDEN = 32

NC = 2
NS = 16
NW = NC * NS
CHUNK = 500
EDGE_ROWS = N_EDGES // CHUNK
ROWS_PER_TILE = EDGE_ROWS // NW
N_PAD = 10240
NODE_SLICE = N_PAD // NS
DEG_W = 8

_MESH = plsc.VectorSubcoreMesh(core_axis_name="c", subcore_axis_name="s")


GRP = 2


@functools.partial(
    pl.kernel,
    mesh=_MESH,
    out_type=jax.ShapeDtypeStruct((NC, N_PAD, HIDDEN), jnp.float32),
    scratch_types=[
        pltpu.VMEM((ROWS_PER_TILE, CHUNK), jnp.int32),
        pltpu.VMEM((ROWS_PER_TILE, CHUNK), jnp.int32),
        pltpu.VMEM((2 * GRP, CHUNK, HIDDEN), jnp.float32),
        pltpu.VMEM_SHARED((N_PAD, HIDDEN), jnp.float32),
        pltpu.SemaphoreType.DMA,
        pltpu.SemaphoreType.DMA,
    ],
    compiler_params=pltpu.CompilerParams(use_tc_tiling_on_sc=False),
)
def _sc_aggregate(edges_hbm, hs_hbm, zero_hbm, out_hbm,
                  src_v, dst_v, rows_v, acc, gsem, ssem):
    c = lax.axis_index("c")
    s = lax.axis_index("s")
    w = s * NC + c
    pltpu.sync_copy(edges_hbm.at[0, pl.ds(w * ROWS_PER_TILE, ROWS_PER_TILE)], src_v)
    pltpu.sync_copy(edges_hbm.at[1, pl.ds(w * ROWS_PER_TILE, ROWS_PER_TILE)], dst_v)
    pltpu.sync_copy(zero_hbm, acc.at[pl.ds(s * NODE_SLICE, NODE_SLICE)])
    plsc.subcore_barrier()

    def gather(j, b):
        pltpu.async_copy(hs_hbm.at[src_v.at[j]], rows_v.at[b], gsem)

    def gather_wait(b):
        pltpu.make_async_copy(hs_hbm.at[src_v.at[0]], rows_v.at[b], gsem).wait()

    def scatter(j, b):
        pltpu.async_copy(rows_v.at[b], acc.at[dst_v.at[j]], ssem, add=True)

    def scatter_wait(b):
        pltpu.make_async_copy(rows_v.at[b], acc.at[dst_v.at[0]], ssem).wait()

    for b in range(GRP):
        gather(b, b)

    def body(g, carry):
        j0 = g * 2 * GRP
        for b in range(GRP):
            gather_wait(b)
        @pl.when(g >= 1)
        def _():
            for b in range(GRP):
                scatter_wait(GRP + b)
        for b in range(GRP):
            gather(j0 + GRP + b, GRP + b)
        for b in range(GRP):
            scatter(j0 + b, b)
        for b in range(GRP):
            gather_wait(GRP + b)
        for b in range(GRP):
            scatter_wait(b)
        @pl.when(g < (ROWS_PER_TILE // (2 * GRP)) - 1)
        def _():
            for b in range(GRP):
                gather(j0 + 2 * GRP + b, b)
        for b in range(GRP):
            scatter(j0 + GRP + b, GRP + b)
        return carry

    lax.fori_loop(0, ROWS_PER_TILE // (2 * GRP), body, 0)
    for b in range(GRP):
        scatter_wait(GRP + b)
    plsc.subcore_barrier()
    pltpu.sync_copy(acc.at[pl.ds(s * NODE_SLICE, NODE_SLICE)],
                    out_hbm.at[c, pl.ds(s * NODE_SLICE, NODE_SLICE)])


@functools.partial(
    pl.kernel,
    mesh=_MESH,
    out_type=jax.ShapeDtypeStruct((NC, N_PAD, DEG_W), jnp.float32),
    scratch_types=[
        pltpu.VMEM((ROWS_PER_TILE, CHUNK), jnp.int32),
        pltpu.VMEM((CHUNK, DEG_W), jnp.float32),
        pltpu.VMEM_SHARED((N_PAD, DEG_W), jnp.float32),
        pltpu.SemaphoreType.DMA,
    ],
    compiler_params=pltpu.CompilerParams(use_tc_tiling_on_sc=False),
)
def _sc_degree(edges_hbm, ones_hbm, zero_hbm, out_hbm, dst_v, ones_v, acc, ssem):
    c = lax.axis_index("c")
    s = lax.axis_index("s")
    w = s * NC + c
    pltpu.sync_copy(edges_hbm.at[1, pl.ds(w * ROWS_PER_TILE, ROWS_PER_TILE)], dst_v)
    pltpu.sync_copy(ones_hbm, ones_v)
    pltpu.sync_copy(zero_hbm, acc.at[pl.ds(s * NODE_SLICE, NODE_SLICE)])
    plsc.subcore_barrier()

    def body(g, carry):
        @pl.when(g >= 1)
        def _():
            for b in range(4):
                pltpu.make_async_copy(ones_v, acc.at[dst_v.at[0]], ssem).wait()
        for b in range(4):
            pltpu.async_copy(ones_v, acc.at[dst_v.at[g * 4 + b]], ssem, add=True)
        return carry

    lax.fori_loop(0, ROWS_PER_TILE // 4, body, 0)
    for b in range(4):
        pltpu.make_async_copy(ones_v, acc.at[dst_v.at[0]], ssem).wait()
    plsc.subcore_barrier()
    pltpu.sync_copy(acc.at[pl.ds(s * NODE_SLICE, NODE_SLICE)],
                    out_hbm.at[c, pl.ds(s * NODE_SLICE, NODE_SLICE)])


N_ROWS = N_NODES // 4
P_ROWS = N_PAD // 4


def _tc_first_body(x4_ref, w1_ref, dege_ref, hs_ref, dinv_ref):
    dinv = lax.rsqrt(dege_ref[0] + dege_ref[1] + 1.0)
    dinv_ref[...] = dinv
    x4 = x4_ref[...]
    for a in range(4):
        h = jnp.dot(x4[:, a, :], w1_ref[...], preferred_element_type=jnp.float32)
        hs_ref[:, 32 * a:32 * (a + 1)] = h * dinv[0:N_ROWS, 32 * a:32 * (a + 1)]


def _tc_mid_body(agg_ref, hs_ref, dinv_ref, b_ref, w_ref, out_ref):
    dinv = dinv_ref[0:N_ROWS]
    h = (agg_ref[0, 0:N_ROWS] + agg_ref[1, 0:N_ROWS] + hs_ref[...]) * dinv
    h = jnp.maximum(h + b_ref[...], 0.0)
    out_ref[...] = jnp.dot(h, w_ref[...], preferred_element_type=jnp.float32) * dinv


def _tc_head_body(agg_ref, hs_ref, dinv_ref, b_ref, wout_ref, bout_ref, out_ref):
    dinv = dinv_ref[0:N_ROWS]
    h = (agg_ref[0, 0:N_ROWS] + agg_ref[1, 0:N_ROWS] + hs_ref[...]) * dinv
    h = jnp.maximum(h + b_ref[...], 0.0)
    t = jnp.dot(h, wout_ref[...], preferred_element_type=jnp.float32) + bout_ref[0, 0]
    out_ref[...] = jnp.maximum(t, 0.0) + jnp.log1p(jnp.exp(-jnp.abs(t))) + 0.001


def kernel(x, edge_index, W1, b1, W2, b2, Wout, bout):
    edges = edge_index.astype(jnp.int32).reshape(2, EDGE_ROWS, CHUNK)
    zero_h = jnp.zeros((NODE_SLICE, HIDDEN), jnp.float32)
    zero_d = jnp.zeros((NODE_SLICE, DEG_W), jnp.float32)
    ones_d = jnp.ones((CHUNK, DEG_W), jnp.float32)
    eye4 = jnp.eye(4, dtype=jnp.float32)
    w2big = jnp.kron(eye4, W2)
    woutbig = jnp.kron(eye4, Wout)
    b1tile = jnp.tile(b1, 4).reshape(1, 128)
    b2tile = jnp.tile(b2, 4).reshape(1, 128)
    x4 = x.reshape(N_ROWS, 4, IN_DIM)

    degp = _sc_degree(edges, ones_d, zero_d)
    dege = jnp.broadcast_to(
        degp[:, :, 0].reshape(NC, P_ROWS, 4, 1), (NC, P_ROWS, 4, HIDDEN)
    ).reshape(NC, P_ROWS, 128)

    hs1p, dinvp = pl.pallas_call(
        _tc_first_body,
        out_shape=(
            jax.ShapeDtypeStruct((N_ROWS, 128), jnp.float32),
            jax.ShapeDtypeStruct((P_ROWS, 128), jnp.float32),
        ),
    )(x4, W1, dege)

    agg1 = _sc_aggregate(edges, hs1p.reshape(N_NODES, HIDDEN), zero_h)
    hs2p = pl.pallas_call(
        _tc_mid_body,
        out_shape=jax.ShapeDtypeStruct((N_ROWS, 128), jnp.float32),
    )(agg1.reshape(NC, P_ROWS, 128), hs1p, dinvp, b1tile, w2big)

    agg2 = _sc_aggregate(edges, hs2p.reshape(N_NODES, HIDDEN), zero_h)
    outp = pl.pallas_call(
        _tc_head_body,
        out_shape=jax.ShapeDtypeStruct((N_ROWS, 4), jnp.float32),
    )(agg2.reshape(NC, P_ROWS, 128), hs2p, dinvp, b2tile, woutbig,
      bout.reshape(1, 1))

    return outp.reshape(N_NODES)


# speedup vs baseline: 1.0125x; 1.0125x over previous
import functools

import jax
import jax.numpy as jnp
from jax import lax
from jax.experimental import pallas as pl
from jax.experimental.pallas import tpu as pltpu
from jax.experimental.pallas import tpu_sc as plsc

N_NODES = 10000
N_EDGES = 320000
IN_DIM = 128
HIDDEN = 32

NC = 2
NS = 16
NW = NC * NS
CHUNK = 500
EDGE_ROWS = N_EDGES // CHUNK
ROWS_PER_TILE = EDGE_ROWS // NW
N_PAD = 10240
NODE_SLICE = N_PAD // NS
DEG_W = 8

_MESH = plsc.VectorSubcoreMesh(core_axis_name="c", subcore_axis_name="s")


GRP = 2


@functools.partial(
    pl.kernel,
    mesh=_MESH,
    out_type=jax.ShapeDtypeStruct((NC, N_PAD, HIDDEN), jnp.float32),
    scratch_types=[
        pltpu.VMEM((ROWS_PER_TILE, CHUNK), jnp.int32),
        pltpu.VMEM((ROWS_PER_TILE, CHUNK), jnp.int32),
        pltpu.VMEM((2 * GRP, CHUNK, HIDDEN), jnp.float32),
        pltpu.VMEM_SHARED((N_PAD, HIDDEN), jnp.float32),
        pltpu.SemaphoreType.DMA,
        pltpu.SemaphoreType.DMA,
    ],
    compiler_params=pltpu.CompilerParams(use_tc_tiling_on_sc=False),
)
def _sc_aggregate(edges_hbm, hs_hbm, zero_hbm, out_hbm,
                  src_v, dst_v, rows_v, acc, gsem, ssem):
    c = lax.axis_index("c")
    s = lax.axis_index("s")
    w = s * NC + c
    pltpu.sync_copy(edges_hbm.at[0, pl.ds(w * ROWS_PER_TILE, ROWS_PER_TILE)], src_v)
    pltpu.sync_copy(edges_hbm.at[1, pl.ds(w * ROWS_PER_TILE, ROWS_PER_TILE)], dst_v)
    pltpu.sync_copy(zero_hbm, acc.at[pl.ds(s * NODE_SLICE, NODE_SLICE)])
    plsc.subcore_barrier()

    def gather(j, b):
        pltpu.async_copy(hs_hbm.at[src_v.at[j]], rows_v.at[b], gsem)

    def gather_wait(b):
        pltpu.make_async_copy(hs_hbm.at[src_v.at[0]], rows_v.at[b], gsem).wait()

    def scatter(j, b):
        pltpu.async_copy(rows_v.at[b], acc.at[dst_v.at[j]], ssem, add=True)

    def scatter_wait(b):
        pltpu.make_async_copy(rows_v.at[b], acc.at[dst_v.at[0]], ssem).wait()

    for b in range(GRP):
        gather(b, b)

    def body(g, carry):
        j0 = g * 2 * GRP
        for b in range(GRP):
            gather_wait(b)
        @pl.when(g >= 1)
        def _():
            for b in range(GRP):
                scatter_wait(GRP + b)
        for b in range(GRP):
            gather(j0 + GRP + b, GRP + b)
        for b in range(GRP):
            scatter(j0 + b, b)
        for b in range(GRP):
            gather_wait(GRP + b)
        for b in range(GRP):
            scatter_wait(b)
        @pl.when(g < (ROWS_PER_TILE // (2 * GRP)) - 1)
        def _():
            for b in range(GRP):
                gather(j0 + 2 * GRP + b, b)
        for b in range(GRP):
            scatter(j0 + GRP + b, GRP + b)
        return carry

    lax.fori_loop(0, ROWS_PER_TILE // (2 * GRP), body, 0)
    for b in range(GRP):
        scatter_wait(GRP + b)
    plsc.subcore_barrier()
    pltpu.sync_copy(acc.at[pl.ds(s * NODE_SLICE, NODE_SLICE)],
                    out_hbm.at[c, pl.ds(s * NODE_SLICE, NODE_SLICE)])


@functools.partial(
    pl.kernel,
    mesh=_MESH,
    out_type=jax.ShapeDtypeStruct((NC, N_PAD, DEG_W), jnp.float32),
    scratch_types=[
        pltpu.VMEM((ROWS_PER_TILE, CHUNK), jnp.int32),
        pltpu.VMEM((CHUNK, DEG_W), jnp.float32),
        pltpu.VMEM_SHARED((N_PAD, DEG_W), jnp.float32),
        pltpu.SemaphoreType.DMA,
    ],
    compiler_params=pltpu.CompilerParams(use_tc_tiling_on_sc=False),
)
def _sc_degree(edges_hbm, ones_hbm, zero_hbm, out_hbm, dst_v, ones_v, acc, ssem):
    c = lax.axis_index("c")
    s = lax.axis_index("s")
    w = s * NC + c
    pltpu.sync_copy(edges_hbm.at[1, pl.ds(w * ROWS_PER_TILE, ROWS_PER_TILE)], dst_v)
    pltpu.sync_copy(ones_hbm, ones_v)
    pltpu.sync_copy(zero_hbm, acc.at[pl.ds(s * NODE_SLICE, NODE_SLICE)])
    plsc.subcore_barrier()

    def body(g, carry):
        @pl.when(g >= 1)
        def _():
            for b in range(4):
                pltpu.make_async_copy(ones_v, acc.at[dst_v.at[0]], ssem).wait()
        for b in range(4):
            pltpu.async_copy(ones_v, acc.at[dst_v.at[g * 4 + b]], ssem, add=True)
        return carry

    lax.fori_loop(0, ROWS_PER_TILE // 4, body, 0)
    for b in range(4):
        pltpu.make_async_copy(ones_v, acc.at[dst_v.at[0]], ssem).wait()
    plsc.subcore_barrier()
    pltpu.sync_copy(acc.at[pl.ds(s * NODE_SLICE, NODE_SLICE)],
                    out_hbm.at[c, pl.ds(s * NODE_SLICE, NODE_SLICE)])


N_ROWS = N_NODES // 4
P_ROWS = N_PAD // 4


def _tc_mm1_body(xp_ref, w1_ref, h_ref):
    h_ref[...] = jnp.dot(xp_ref[...], w1_ref[...],
                         preferred_element_type=jnp.float32)


def _tc_scale_body(h_ref, dege_ref, hs_ref, dinv_ref):
    dinv = lax.rsqrt(dege_ref[0] + dege_ref[1] + 1.0)
    dinv_ref[...] = dinv
    hs_ref[...] = h_ref[...] * dinv[0:N_ROWS]


def _tc_mid_body(agg_ref, hs_ref, dinv_ref, b_ref, w_ref, out_ref):
    dinv = dinv_ref[0:N_ROWS]
    h = (agg_ref[0, 0:N_ROWS] + agg_ref[1, 0:N_ROWS] + hs_ref[...]) * dinv
    h = jnp.maximum(h + b_ref[...], 0.0)
    out_ref[...] = jnp.dot(h, w_ref[...], preferred_element_type=jnp.float32) * dinv


def _tc_head_body(agg_ref, hs_ref, dinv_ref, b_ref, wout_ref, bout_ref, out_ref):
    dinv = dinv_ref[0:N_ROWS]
    h = (agg_ref[0, 0:N_ROWS] + agg_ref[1, 0:N_ROWS] + hs_ref[...]) * dinv
    h = jnp.maximum(h + b_ref[...], 0.0)
    t = jnp.dot(h, wout_ref[...], preferred_element_type=jnp.float32) + bout_ref[0, 0]
    out_ref[...] = jnp.maximum(t, 0.0) + jnp.log1p(jnp.exp(-jnp.abs(t))) + 0.001


def kernel(x, edge_index, W1, b1, W2, b2, Wout, bout):
    edges = edge_index.astype(jnp.int32).reshape(2, EDGE_ROWS, CHUNK)
    zero_h = jnp.zeros((NODE_SLICE, HIDDEN), jnp.float32)
    zero_d = jnp.zeros((NODE_SLICE, DEG_W), jnp.float32)
    ones_d = jnp.ones((CHUNK, DEG_W), jnp.float32)
    eye4 = jnp.eye(4, dtype=jnp.float32)
    w1big = jnp.kron(eye4, W1)
    w2big = jnp.kron(eye4, W2)
    woutbig = jnp.kron(eye4, Wout)
    b1tile = jnp.tile(b1, 4).reshape(1, 128)
    b2tile = jnp.tile(b2, 4).reshape(1, 128)
    xp = x.reshape(N_ROWS, 4 * IN_DIM)

    degp = _sc_degree(edges, ones_d, zero_d)
    dege = jnp.broadcast_to(
        degp[:, :, 0].reshape(NC, P_ROWS, 4, 1), (NC, P_ROWS, 4, HIDDEN)
    ).reshape(NC, P_ROWS, 128)

    h1p = pl.pallas_call(
        _tc_mm1_body,
        out_shape=jax.ShapeDtypeStruct((N_ROWS, 128), jnp.float32),
    )(xp, w1big)
    hs1p, dinvp = pl.pallas_call(
        _tc_scale_body,
        out_shape=(
            jax.ShapeDtypeStruct((N_ROWS, 128), jnp.float32),
            jax.ShapeDtypeStruct((P_ROWS, 128), jnp.float32),
        ),
    )(h1p, dege)

    agg1 = _sc_aggregate(edges, hs1p.reshape(N_NODES, HIDDEN), zero_h)
    hs2p = pl.pallas_call(
        _tc_mid_body,
        out_shape=jax.ShapeDtypeStruct((N_ROWS, 128), jnp.float32),
    )(agg1.reshape(NC, P_ROWS, 128), hs1p, dinvp, b1tile, w2big)

    agg2 = _sc_aggregate(edges, hs2p.reshape(N_NODES, HIDDEN), zero_h)
    outp = pl.pallas_call(
        _tc_head_body,
        out_shape=jax.ShapeDtypeStruct((N_ROWS, 4), jnp.float32),
    )(agg2.reshape(NC, P_ROWS, 128), hs2p, dinvp, b2tile, woutbig,
      bout.reshape(1, 1))

    return outp.reshape(N_NODES)


# speedup vs baseline: 1.0347x; 1.0220x over previous
import functools

import jax
import jax.numpy as jnp
from jax import lax
from jax.experimental import pallas as pl
from jax.experimental.pallas import tpu as pltpu
from jax.experimental.pallas import tpu_sc as plsc

N_NODES = 10000
N_EDGES = 320000
IN_DIM = 128
HIDDEN = 32

NC = 2
NS = 16
NW = NC * NS
CHUNK = 500
EDGE_ROWS = N_EDGES // CHUNK
ROWS_PER_TILE = EDGE_ROWS // NW
N_PAD = 10240
NODE_SLICE = N_PAD // NS
DEG_W = 8

_MESH = plsc.VectorSubcoreMesh(core_axis_name="c", subcore_axis_name="s")


GRP = 2


@functools.partial(
    pl.kernel,
    mesh=_MESH,
    out_type=jax.ShapeDtypeStruct((NC, N_PAD, HIDDEN), jnp.float32),
    scratch_types=[
        pltpu.VMEM((ROWS_PER_TILE, CHUNK), jnp.int32),
        pltpu.VMEM((ROWS_PER_TILE, CHUNK), jnp.int32),
        pltpu.VMEM((2 * GRP, CHUNK, HIDDEN), jnp.float32),
        pltpu.VMEM_SHARED((N_PAD, HIDDEN), jnp.float32),
        pltpu.SemaphoreType.DMA,
        pltpu.SemaphoreType.DMA,
    ],
    compiler_params=pltpu.CompilerParams(use_tc_tiling_on_sc=False),
)
def _sc_aggregate(edges_hbm, hs_hbm, zero_hbm, out_hbm,
                  src_v, dst_v, rows_v, acc, gsem, ssem):
    c = lax.axis_index("c")
    s = lax.axis_index("s")
    w = s * NC + c
    pltpu.async_copy(edges_hbm.at[0, pl.ds(w * ROWS_PER_TILE, ROWS_PER_TILE)],
                     src_v, gsem)
    pltpu.async_copy(edges_hbm.at[1, pl.ds(w * ROWS_PER_TILE, ROWS_PER_TILE)],
                     dst_v, gsem)
    pltpu.async_copy(zero_hbm, acc.at[pl.ds(s * NODE_SLICE, NODE_SLICE)], gsem)
    pltpu.make_async_copy(edges_hbm.at[0, pl.ds(0, ROWS_PER_TILE)], src_v, gsem).wait()
    pltpu.make_async_copy(edges_hbm.at[1, pl.ds(0, ROWS_PER_TILE)], dst_v, gsem).wait()
    pltpu.make_async_copy(zero_hbm, acc.at[pl.ds(0, NODE_SLICE)], gsem).wait()
    plsc.subcore_barrier()

    def gather(j, b):
        pltpu.async_copy(hs_hbm.at[src_v.at[j]], rows_v.at[b], gsem)

    def gather_wait(b):
        pltpu.make_async_copy(hs_hbm.at[src_v.at[0]], rows_v.at[b], gsem).wait()

    def scatter(j, b):
        pltpu.async_copy(rows_v.at[b], acc.at[dst_v.at[j]], ssem, add=True)

    def scatter_wait(b):
        pltpu.make_async_copy(rows_v.at[b], acc.at[dst_v.at[0]], ssem).wait()

    for b in range(GRP):
        gather(b, b)

    def body(g, carry):
        j0 = g * 2 * GRP
        for b in range(GRP):
            gather_wait(b)
        @pl.when(g >= 1)
        def _():
            for b in range(GRP):
                scatter_wait(GRP + b)
        for b in range(GRP):
            gather(j0 + GRP + b, GRP + b)
        for b in range(GRP):
            scatter(j0 + b, b)
        for b in range(GRP):
            gather_wait(GRP + b)
        for b in range(GRP):
            scatter_wait(b)
        @pl.when(g < (ROWS_PER_TILE // (2 * GRP)) - 1)
        def _():
            for b in range(GRP):
                gather(j0 + 2 * GRP + b, b)
        for b in range(GRP):
            scatter(j0 + GRP + b, GRP + b)
        return carry

    lax.fori_loop(0, ROWS_PER_TILE // (2 * GRP), body, 0)
    for b in range(GRP):
        scatter_wait(GRP + b)
    plsc.subcore_barrier()
    pltpu.sync_copy(acc.at[pl.ds(s * NODE_SLICE, NODE_SLICE)],
                    out_hbm.at[c, pl.ds(s * NODE_SLICE, NODE_SLICE)])


@functools.partial(
    pl.kernel,
    mesh=_MESH,
    out_type=jax.ShapeDtypeStruct((NC, N_PAD, DEG_W), jnp.float32),
    scratch_types=[
        pltpu.VMEM((ROWS_PER_TILE, CHUNK), jnp.int32),
        pltpu.VMEM((CHUNK, DEG_W), jnp.float32),
        pltpu.VMEM_SHARED((N_PAD, DEG_W), jnp.float32),
        pltpu.SemaphoreType.DMA,
    ],
    compiler_params=pltpu.CompilerParams(use_tc_tiling_on_sc=False),
)
def _sc_degree(edges_hbm, ones_hbm, zero_hbm, out_hbm, dst_v, ones_v, acc, ssem):
    c = lax.axis_index("c")
    s = lax.axis_index("s")
    w = s * NC + c
    pltpu.async_copy(edges_hbm.at[1, pl.ds(w * ROWS_PER_TILE, ROWS_PER_TILE)],
                     dst_v, ssem)
    pltpu.async_copy(ones_hbm, ones_v, ssem)
    pltpu.async_copy(zero_hbm, acc.at[pl.ds(s * NODE_SLICE, NODE_SLICE)], ssem)
    pltpu.make_async_copy(edges_hbm.at[1, pl.ds(0, ROWS_PER_TILE)], dst_v, ssem).wait()
    pltpu.make_async_copy(ones_hbm, ones_v, ssem).wait()
    pltpu.make_async_copy(zero_hbm, acc.at[pl.ds(0, NODE_SLICE)], ssem).wait()
    plsc.subcore_barrier()

    def body(g, carry):
        @pl.when(g >= 1)
        def _():
            for b in range(4):
                pltpu.make_async_copy(ones_v, acc.at[dst_v.at[0]], ssem).wait()
        for b in range(4):
            pltpu.async_copy(ones_v, acc.at[dst_v.at[g * 4 + b]], ssem, add=True)
        return carry

    lax.fori_loop(0, ROWS_PER_TILE // 4, body, 0)
    for b in range(4):
        pltpu.make_async_copy(ones_v, acc.at[dst_v.at[0]], ssem).wait()
    plsc.subcore_barrier()
    pltpu.sync_copy(acc.at[pl.ds(s * NODE_SLICE, NODE_SLICE)],
                    out_hbm.at[c, pl.ds(s * NODE_SLICE, NODE_SLICE)])


N_ROWS = N_NODES // 4
P_ROWS = N_PAD // 4


def _tc_mm1_body(xp_ref, w1_ref, h_ref):
    h_ref[...] = jnp.dot(xp_ref[...], w1_ref[...],
                         preferred_element_type=jnp.float32)


def _tc_scale_body(h_ref, dege_ref, hs_ref, dinv_ref):
    dinv = lax.rsqrt(dege_ref[0] + dege_ref[1] + 1.0)
    dinv_ref[...] = dinv
    hs_ref[...] = h_ref[...] * dinv[0:N_ROWS]


def _tc_mid_body(agg_ref, hs_ref, dinv_ref, b_ref, w_ref, out_ref):
    dinv = dinv_ref[0:N_ROWS]
    h = (agg_ref[0, 0:N_ROWS] + agg_ref[1, 0:N_ROWS] + hs_ref[...]) * dinv
    h = jnp.maximum(h + b_ref[...], 0.0)
    out_ref[...] = jnp.dot(h, w_ref[...], preferred_element_type=jnp.float32) * dinv


def _tc_head_body(agg_ref, hs_ref, dinv_ref, b_ref, wout_ref, bout_ref, out_ref):
    dinv = dinv_ref[0:N_ROWS]
    h = (agg_ref[0, 0:N_ROWS] + agg_ref[1, 0:N_ROWS] + hs_ref[...]) * dinv
    h = jnp.maximum(h + b_ref[...], 0.0)
    t = jnp.dot(h, wout_ref[...], preferred_element_type=jnp.float32) + bout_ref[0, 0]
    out_ref[...] = jnp.maximum(t, 0.0) + jnp.log1p(jnp.exp(-jnp.abs(t))) + 0.001


def kernel(x, edge_index, W1, b1, W2, b2, Wout, bout):
    edges = edge_index.astype(jnp.int32).reshape(2, EDGE_ROWS, CHUNK)
    zero_h = jnp.zeros((NODE_SLICE, HIDDEN), jnp.float32)
    zero_d = jnp.zeros((NODE_SLICE, DEG_W), jnp.float32)
    ones_d = jnp.ones((CHUNK, DEG_W), jnp.float32)
    eye4 = jnp.eye(4, dtype=jnp.float32)
    w1big = jnp.kron(eye4, W1)
    w2big = jnp.kron(eye4, W2)
    woutbig = jnp.kron(eye4, Wout)
    b1tile = jnp.tile(b1, 4).reshape(1, 128)
    b2tile = jnp.tile(b2, 4).reshape(1, 128)
    xp = x.reshape(N_ROWS, 4 * IN_DIM)

    degp = _sc_degree(edges, ones_d, zero_d)
    dege = jnp.broadcast_to(
        degp[:, :, 0].reshape(NC, P_ROWS, 4, 1), (NC, P_ROWS, 4, HIDDEN)
    ).reshape(NC, P_ROWS, 128)

    h1p = pl.pallas_call(
        _tc_mm1_body,
        out_shape=jax.ShapeDtypeStruct((N_ROWS, 128), jnp.float32),
    )(xp, w1big)
    hs1p, dinvp = pl.pallas_call(
        _tc_scale_body,
        out_shape=(
            jax.ShapeDtypeStruct((N_ROWS, 128), jnp.float32),
            jax.ShapeDtypeStruct((P_ROWS, 128), jnp.float32),
        ),
    )(h1p, dege)

    agg1 = _sc_aggregate(edges, hs1p.reshape(N_NODES, HIDDEN), zero_h)
    hs2p = pl.pallas_call(
        _tc_mid_body,
        out_shape=jax.ShapeDtypeStruct((N_ROWS, 128), jnp.float32),
    )(agg1.reshape(NC, P_ROWS, 128), hs1p, dinvp, b1tile, w2big)

    agg2 = _sc_aggregate(edges, hs2p.reshape(N_NODES, HIDDEN), zero_h)
    outp = pl.pallas_call(
        _tc_head_body,
        out_shape=jax.ShapeDtypeStruct((N_ROWS, 4), jnp.float32),
    )(agg2.reshape(NC, P_ROWS, 128), hs2p, dinvp, b2tile, woutbig,
      bout.reshape(1, 1))

    return outp.reshape(N_NODES)


# speedup vs baseline: 1.0524x; 1.0171x over previous
import functools

import jax
import jax.numpy as jnp
from jax import lax
from jax.experimental import pallas as pl
from jax.experimental.pallas import tpu as pltpu
from jax.experimental.pallas import tpu_sc as plsc

N_NODES = 10000
N_EDGES = 320000
IN_DIM = 128
HIDDEN = 32

NC = 2
NS = 16
NW = NC * NS
CHUNK = 500
EDGE_ROWS = N_EDGES // CHUNK
ROWS_PER_TILE = EDGE_ROWS // NW
N_PAD = 10240
NODE_SLICE = N_PAD // NS
DEG_W = 8

_MESH = plsc.VectorSubcoreMesh(core_axis_name="c", subcore_axis_name="s")


GRP = 2


@functools.partial(
    pl.kernel,
    mesh=_MESH,
    out_type=jax.ShapeDtypeStruct((NC, N_PAD, HIDDEN), jnp.float32),
    scratch_types=[
        pltpu.VMEM((ROWS_PER_TILE, CHUNK), jnp.int32),
        pltpu.VMEM((ROWS_PER_TILE, CHUNK), jnp.int32),
        pltpu.VMEM((2 * GRP, CHUNK, HIDDEN), jnp.float32),
        pltpu.VMEM_SHARED((N_PAD, HIDDEN), jnp.float32),
        pltpu.SemaphoreType.DMA,
        pltpu.SemaphoreType.DMA,
    ],
    compiler_params=pltpu.CompilerParams(use_tc_tiling_on_sc=False),
)
def _sc_aggregate(edges_hbm, hs_hbm, zero_hbm, out_hbm,
                  src_v, dst_v, rows_v, acc, gsem, ssem):
    c = lax.axis_index("c")
    s = lax.axis_index("s")
    w = s * NC + c
    pltpu.async_copy(edges_hbm.at[0, pl.ds(w * ROWS_PER_TILE, ROWS_PER_TILE)],
                     src_v, gsem)
    pltpu.async_copy(edges_hbm.at[1, pl.ds(w * ROWS_PER_TILE, ROWS_PER_TILE)],
                     dst_v, gsem)
    pltpu.async_copy(zero_hbm, acc.at[pl.ds(s * NODE_SLICE, NODE_SLICE)], gsem)
    pltpu.make_async_copy(edges_hbm.at[0, pl.ds(0, ROWS_PER_TILE)], src_v, gsem).wait()
    pltpu.make_async_copy(edges_hbm.at[1, pl.ds(0, ROWS_PER_TILE)], dst_v, gsem).wait()
    pltpu.make_async_copy(zero_hbm, acc.at[pl.ds(0, NODE_SLICE)], gsem).wait()

    def gather(j, b):
        pltpu.async_copy(hs_hbm.at[src_v.at[j]], rows_v.at[b], gsem)

    def gather_wait(b):
        pltpu.make_async_copy(hs_hbm.at[src_v.at[0]], rows_v.at[b], gsem).wait()

    def scatter(j, b):
        pltpu.async_copy(rows_v.at[b], acc.at[dst_v.at[j]], ssem, add=True)

    def scatter_wait(b):
        pltpu.make_async_copy(rows_v.at[b], acc.at[dst_v.at[0]], ssem).wait()

    for b in range(GRP):
        gather(b, b)
    plsc.subcore_barrier()

    def body(g, carry):
        j0 = g * 2 * GRP
        for b in range(GRP):
            gather_wait(b)
        @pl.when(g >= 1)
        def _():
            for b in range(GRP):
                scatter_wait(GRP + b)
        for b in range(GRP):
            gather(j0 + GRP + b, GRP + b)
        for b in range(GRP):
            scatter(j0 + b, b)
        for b in range(GRP):
            gather_wait(GRP + b)
        for b in range(GRP):
            scatter_wait(b)
        @pl.when(g < (ROWS_PER_TILE // (2 * GRP)) - 1)
        def _():
            for b in range(GRP):
                gather(j0 + 2 * GRP + b, b)
        for b in range(GRP):
            scatter(j0 + GRP + b, GRP + b)
        return carry

    lax.fori_loop(0, ROWS_PER_TILE // (2 * GRP), body, 0)
    for b in range(GRP):
        scatter_wait(GRP + b)
    plsc.subcore_barrier()
    pltpu.sync_copy(acc.at[pl.ds(s * NODE_SLICE, NODE_SLICE)],
                    out_hbm.at[c, pl.ds(s * NODE_SLICE, NODE_SLICE)])


@functools.partial(
    pl.kernel,
    mesh=_MESH,
    out_type=jax.ShapeDtypeStruct((NC, N_PAD, DEG_W), jnp.float32),
    scratch_types=[
        pltpu.VMEM((ROWS_PER_TILE, CHUNK), jnp.int32),
        pltpu.VMEM((CHUNK, DEG_W), jnp.float32),
        pltpu.VMEM_SHARED((N_PAD, DEG_W), jnp.float32),
        pltpu.SemaphoreType.DMA,
    ],
    compiler_params=pltpu.CompilerParams(use_tc_tiling_on_sc=False),
)
def _sc_degree(edges_hbm, ones_hbm, zero_hbm, out_hbm, dst_v, ones_v, acc, ssem):
    c = lax.axis_index("c")
    s = lax.axis_index("s")
    w = s * NC + c
    pltpu.async_copy(edges_hbm.at[1, pl.ds(w * ROWS_PER_TILE, ROWS_PER_TILE)],
                     dst_v, ssem)
    pltpu.async_copy(ones_hbm, ones_v, ssem)
    pltpu.async_copy(zero_hbm, acc.at[pl.ds(s * NODE_SLICE, NODE_SLICE)], ssem)
    pltpu.make_async_copy(edges_hbm.at[1, pl.ds(0, ROWS_PER_TILE)], dst_v, ssem).wait()
    pltpu.make_async_copy(ones_hbm, ones_v, ssem).wait()
    pltpu.make_async_copy(zero_hbm, acc.at[pl.ds(0, NODE_SLICE)], ssem).wait()
    plsc.subcore_barrier()

    def body(g, carry):
        @pl.when(g >= 1)
        def _():
            for b in range(4):
                pltpu.make_async_copy(ones_v, acc.at[dst_v.at[0]], ssem).wait()
        for b in range(4):
            pltpu.async_copy(ones_v, acc.at[dst_v.at[g * 4 + b]], ssem, add=True)
        return carry

    lax.fori_loop(0, ROWS_PER_TILE // 4, body, 0)
    for b in range(4):
        pltpu.make_async_copy(ones_v, acc.at[dst_v.at[0]], ssem).wait()
    plsc.subcore_barrier()
    pltpu.sync_copy(acc.at[pl.ds(s * NODE_SLICE, NODE_SLICE)],
                    out_hbm.at[c, pl.ds(s * NODE_SLICE, NODE_SLICE)])


N_ROWS = N_NODES // 4
P_ROWS = N_PAD // 4


def _tc_mm1_body(xp_ref, w1_ref, h_ref):
    h_ref[...] = jnp.dot(xp_ref[...], w1_ref[...],
                         preferred_element_type=jnp.float32)


def _tc_scale_body(h_ref, dege_ref, hs_ref, dinv_ref):
    dinv = lax.rsqrt(dege_ref[0] + dege_ref[1] + 1.0)
    dinv_ref[...] = dinv
    hs_ref[...] = h_ref[...] * dinv[0:N_ROWS]


def _tc_mid_body(agg_ref, hs_ref, dinv_ref, b_ref, w_ref, out_ref):
    dinv = dinv_ref[0:N_ROWS]
    h = (agg_ref[0, 0:N_ROWS] + agg_ref[1, 0:N_ROWS] + hs_ref[...]) * dinv
    h = jnp.maximum(h + b_ref[...], 0.0)
    out_ref[...] = jnp.dot(h, w_ref[...], preferred_element_type=jnp.float32) * dinv


def _tc_head_body(agg_ref, hs_ref, dinv_ref, b_ref, wout_ref, bout_ref, out_ref):
    dinv = dinv_ref[0:N_ROWS]
    h = (agg_ref[0, 0:N_ROWS] + agg_ref[1, 0:N_ROWS] + hs_ref[...]) * dinv
    h = jnp.maximum(h + b_ref[...], 0.0)
    t = jnp.dot(h, wout_ref[...], preferred_element_type=jnp.float32) + bout_ref[0, 0]
    out_ref[...] = jnp.maximum(t, 0.0) + jnp.log1p(jnp.exp(-jnp.abs(t))) + 0.001


def kernel(x, edge_index, W1, b1, W2, b2, Wout, bout):
    edges = edge_index.astype(jnp.int32).reshape(2, EDGE_ROWS, CHUNK)
    zero_h = jnp.zeros((NODE_SLICE, HIDDEN), jnp.float32)
    zero_d = jnp.zeros((NODE_SLICE, DEG_W), jnp.float32)
    ones_d = jnp.ones((CHUNK, DEG_W), jnp.float32)
    eye4 = jnp.eye(4, dtype=jnp.float32)
    w1big = jnp.kron(eye4, W1)
    w2big = jnp.kron(eye4, W2)
    woutbig = jnp.kron(eye4, Wout)
    b1tile = jnp.tile(b1, 4).reshape(1, 128)
    b2tile = jnp.tile(b2, 4).reshape(1, 128)
    xp = x.reshape(N_ROWS, 4 * IN_DIM)

    degp = _sc_degree(edges, ones_d, zero_d)
    dege = jnp.broadcast_to(
        degp[:, :, 0].reshape(NC, P_ROWS, 4, 1), (NC, P_ROWS, 4, HIDDEN)
    ).reshape(NC, P_ROWS, 128)

    h1p = pl.pallas_call(
        _tc_mm1_body,
        out_shape=jax.ShapeDtypeStruct((N_ROWS, 128), jnp.float32),
    )(xp, w1big)
    hs1p, dinvp = pl.pallas_call(
        _tc_scale_body,
        out_shape=(
            jax.ShapeDtypeStruct((N_ROWS, 128), jnp.float32),
            jax.ShapeDtypeStruct((P_ROWS, 128), jnp.float32),
        ),
    )(h1p, dege)

    agg1 = _sc_aggregate(edges, hs1p.reshape(N_NODES, HIDDEN), zero_h)
    hs2p = pl.pallas_call(
        _tc_mid_body,
        out_shape=jax.ShapeDtypeStruct((N_ROWS, 128), jnp.float32),
    )(agg1.reshape(NC, P_ROWS, 128), hs1p, dinvp, b1tile, w2big)

    agg2 = _sc_aggregate(edges, hs2p.reshape(N_NODES, HIDDEN), zero_h)
    outp = pl.pallas_call(
        _tc_head_body,
        out_shape=jax.ShapeDtypeStruct((N_ROWS, 4), jnp.float32),
    )(agg2.reshape(NC, P_ROWS, 128), hs2p, dinvp, b2tile, woutbig,
      bout.reshape(1, 1))

    return outp.reshape(N_NODES)
